# baseline (speedup 1.0000x reference)
import functools

import jax
import jax.numpy as jnp
from jax import lax
from jax.experimental import pallas as pl
from jax.experimental.pallas import tpu as pltpu
from jax.experimental.pallas import tpu_sc as plsc

VOCAB_SIZE = 1000
_NC = 2
_NS = 16
_LANES = 16


def _sc_kernel(n_total, n_words, vocab, bank_pad_w, rows_per_w):
    d = n_words * vocab
    n_chunk = (n_words + _LANES - 1) // _LANES

    def body(bank_hbm, idx_hbm, out_hbm, idx_v, bank_v, buf0, buf1,
             sem0, sem1):
        wid = lax.axis_index("s") * _NC + lax.axis_index("c")
        base = wid * rows_per_w
        cp = pltpu.make_async_copy(
            idx_hbm.at[pl.ds(base, rows_per_w)],
            idx_v.at[pl.ds(_LANES, rows_per_w)], sem0)
        cp.start()
        cp.wait()
        cp = pltpu.make_async_copy(
            bank_hbm, bank_v.at[pl.ds(1, bank_v.shape[0] - 1)], sem1)
        cp.start()
        cp.wait()

        zvec = jnp.zeros((_LANES,), jnp.float32)

        def zbody(r, carry):
            buf0[pl.ds(r * _LANES, _LANES)] = zvec
            buf1[pl.ds(r * _LANES, _LANES)] = zvec
            return carry

        lax.fori_loop(0, d // _LANES, zbody, 0)

        wio = lax.iota(jnp.int32, _LANES)
        ones = jnp.full((_LANES,), 1.0, jnp.float32)
        masks = [(wio + _LANES * c) < n_words for c in range(n_chunk)]
        bufs = (buf0, buf1)
        sems = (sem0, sem1)
        saved_offs = [None, None]

        locs = list(range(rows_per_w))
        for t, local in enumerate(locs):
            p = t % 2
            buf, sem = bufs[p], sems[p]
            if t >= 2:
                pltpu.make_async_copy(
                    buf, out_hbm.at[base + locs[t - 2]], sem).wait()
                for c in range(n_chunk):
                    plsc.store_scatter(
                        buf, [saved_offs[p][c]], zvec, mask=masks[c])
            ksp = plsc.load_gather(
                idx_v, [jnp.full((_LANES,), local + _LANES, jnp.int32)])
            offs = []
            for c in range(n_chunk):
                wv = wio + _LANES * c
                tok = plsc.load_gather(bank_v, [ksp + 1, wv])
                off = wv * vocab + tok
                off = jnp.where(masks[c], off, 0)
                plsc.store_scatter(buf, [off], ones, mask=masks[c])
                offs.append(off)
            saved_offs[p] = offs
            pltpu.make_async_copy(buf, out_hbm.at[base + local], sem).start()
        pltpu.make_async_copy(
            bufs[0], out_hbm.at[base + locs[-2]], sems[0]).wait()
        pltpu.make_async_copy(
            bufs[1], out_hbm.at[base + locs[-1]], sems[1]).wait()

    return body


def kernel(features, reports, sentence_bank, weights):
    B, n_sentences, n_words = reports.shape
    k_first, bank_w = sentence_bank.shape

    key = jax.random.key(42)
    idx = jax.random.categorical(key, jnp.log(weights), shape=(B, n_sentences))
    idx = idx.astype(jnp.int32).reshape(-1)

    bank = sentence_bank.astype(jnp.int32)
    if bank_w < n_words:
        bank = jnp.pad(bank, ((0, 0), (0, n_words - bank_w)))
    bank = bank[:, :n_words]
    pad_w = (-n_words) % _LANES
    bank = jnp.pad(bank, ((0, 0), (0, pad_w)))

    n_total = B * n_sentences
    rows_per_w = n_total // (_NC * _NS)
    d = n_words * VOCAB_SIZE

    mesh = plsc.VectorSubcoreMesh(core_axis_name="c", subcore_axis_name="s")
    sc_call = functools.partial(
        pl.kernel,
        out_type=jax.ShapeDtypeStruct((n_total, d), jnp.float32),
        mesh=mesh,
        compiler_params=pltpu.CompilerParams(needs_layout_passes=False),
        scratch_types=[
            pltpu.VMEM((rows_per_w + _LANES,), jnp.int32),
            pltpu.VMEM((k_first + 1, n_words + pad_w), jnp.int32),
            pltpu.VMEM((d,), jnp.float32),
            pltpu.VMEM((d,), jnp.float32),
            pltpu.SemaphoreType.DMA,
            pltpu.SemaphoreType.DMA,
        ],
    )(_sc_kernel(n_total, n_words, VOCAB_SIZE, n_words + pad_w, rows_per_w))
    out = sc_call(bank, idx)

    out = out.reshape(B, n_sentences, n_words, VOCAB_SIZE)
    stops = jnp.zeros((B, n_sentences), dtype=jnp.float32)
    return (out, stops)


# speedup vs baseline: 1.9172x; 1.9172x over previous
import jax
import jax.numpy as jnp
from jax.experimental import pallas as pl
from jax.experimental.pallas import tpu as pltpu

VOCAB_SIZE = 1000
_NSEM = 8
_WINDOW = 128


def _make_kernel(n_total, n_sentences):
    def _kern(idx_ref, bank_ref, out_ref, oh0_ref, oh1_ref, *sems):
        n_words, k = bank_ref.shape
        vocab = oh0_ref.shape[-1]
        col = jax.lax.broadcasted_iota(jnp.int32, (n_words, vocab), 1)
        bank_i = bank_ref[...].astype(jnp.int32)
        for kk in range(k):
            tok_col = bank_i[:, kk:kk + 1]
            oh = (tok_col == col).astype(jnp.float32)
            oh0_ref[kk] = oh
            oh1_ref[kk] = oh

        ohs = (oh0_ref, oh1_ref)

        def copy(i):
            kk = idx_ref[i // n_sentences, i % n_sentences]
            return pltpu.make_async_copy(
                ohs[i % 2].at[kk], out_ref.at[i], sems[i % _NSEM])

        for i in range(min(_WINDOW, n_total)):
            copy(i).start()
        for i in range(n_total):
            j = i + _WINDOW
            if j < n_total:
                copy(j).start()
            copy(i).wait()

    return _kern


def kernel(features, reports, sentence_bank, weights):
    B, n_sentences, n_words = reports.shape
    k_first, bank_w = sentence_bank.shape

    key = jax.random.key(42)
    idx = jax.random.categorical(key, jnp.log(weights), shape=(B, n_sentences))
    idx = idx.astype(jnp.int32)

    if bank_w < n_words:
        sentence_bank = jnp.pad(sentence_bank, ((0, 0), (0, n_words - bank_w)))
    bank_t = sentence_bank[:, :n_words].T.astype(jnp.float32)

    n_total = B * n_sentences
    out = pl.pallas_call(
        _make_kernel(n_total, n_sentences),
        in_specs=[
            pl.BlockSpec(memory_space=pltpu.SMEM),
            pl.BlockSpec(memory_space=pltpu.VMEM),
        ],
        out_specs=pl.BlockSpec(memory_space=pl.ANY),
        out_shape=jax.ShapeDtypeStruct(
            (n_total, n_words, VOCAB_SIZE), jnp.float32
        ),
        scratch_shapes=(
            [pltpu.VMEM((k_first, n_words, VOCAB_SIZE), jnp.float32),
             pltpu.VMEM((k_first, n_words, VOCAB_SIZE), jnp.float32)]
            + [pltpu.SemaphoreType.DMA] * _NSEM
        ),
    )(idx, bank_t)

    out = out.reshape(B, n_sentences, n_words, VOCAB_SIZE)
    stops = jnp.zeros((B, n_sentences), dtype=jnp.float32)
    return (out, stops)
